# Initial kernel scaffold; baseline (speedup 1.0000x reference)
import functools

import jax
import jax.numpy as jnp
from jax import lax
from jax.experimental import pallas as pl
from jax.experimental.pallas import tpu as pltpu
from jax.experimental.pallas import tpu_sc as plsc

NNODES = 10000
NEDGES = 320000
NEG = 0.2
EPS = 1e-16

NTILES = 32
NE_T = NEDGES // NTILES
K = 125
NCH = NE_T // K
RPT = NNODES // 16


def _lrelu(x):
    return jnp.where(x >= 0, x, NEG * x)


def _tc_head(nprev, wcur, x_is_raw):
    def body(*refs):
        if x_is_raw:
            (x_ref, W_ref, as_ref, ad_ref,
             h_ref, asv_ref, adv_ref, M_ref, sw_ref) = refs
            o = x_ref[...]
        else:
            (acc_ref, hp_ref, swp_ref, W_ref, as_ref, ad_ref,
             h_ref, asv_ref, adv_ref, M_ref, sw_ref) = refs
            accs = acc_ref[0] + acc_ref[1]
            num = accs[:, :nprev] + swp_ref[...] * hp_ref[...]
            den = accs[:, nprev:nprev + 1] + swp_ref[...] + EPS
            o = jnp.maximum(num / den, 0.0)
        h = jnp.dot(o, W_ref[...], preferred_element_type=jnp.float32)
        h_ref[...] = h
        asv = jnp.sum(h * as_ref[...], axis=1, keepdims=True)
        adv = jnp.sum(h * ad_ref[...], axis=1, keepdims=True)
        A = jnp.max(asv)
        M = _lrelu(A + adv)
        asv_ref[...] = asv
        adv_ref[...] = adv
        M_ref[...] = M
        sw_ref[...] = jnp.exp(_lrelu(asv + adv) - M)

    out_shape = [
        jax.ShapeDtypeStruct((NNODES, wcur), jnp.float32),
        jax.ShapeDtypeStruct((NNODES, 1), jnp.float32),
        jax.ShapeDtypeStruct((NNODES, 1), jnp.float32),
        jax.ShapeDtypeStruct((NNODES, 1), jnp.float32),
        jax.ShapeDtypeStruct((NNODES, 1), jnp.float32),
    ]
    return pl.pallas_call(body, out_shape=out_shape)


def _tc_final(acc, h3, sw3, wprev, cout):
    def body(acc_ref, h_ref, sw_ref, out_ref):
        accs = acc_ref[0] + acc_ref[1]
        num = accs[:, :wprev] + sw_ref[...] * h_ref[...]
        den = accs[:, wprev:wprev + 1] + sw_ref[...] + EPS
        out_ref[...] = (num / den)[:, :cout]

    return pl.pallas_call(
        body, out_shape=jax.ShapeDtypeStruct((NNODES, cout), jnp.float32)
    )(acc, h3, sw3)


def _make_sc_layer(w):
    accw = w + 16
    w16 = w // 16
    mesh = plsc.VectorSubcoreMesh(core_axis_name="c", subcore_axis_name="s")

    @functools.partial(
        pl.kernel,
        out_type=jax.ShapeDtypeStruct((2, NNODES, accw), jnp.float32),
        mesh=mesh,
        scratch_types=[
            pltpu.VMEM((NE_T,), jnp.int32),
            pltpu.VMEM((NE_T,), jnp.int32),
            pltpu.VMEM((NCH, K), jnp.int32),
            pltpu.VMEM((NNODES,), jnp.float32),
            pltpu.VMEM((NNODES,), jnp.float32),
            pltpu.VMEM((NNODES,), jnp.float32),
            pltpu.VMEM((NE_T,), jnp.float32),
            pltpu.VMEM((K, w), jnp.float32),
            pltpu.VMEM((K, accw), jnp.float32),
            pltpu.VMEM_SHARED((NNODES, accw), jnp.float32),
            pltpu.SemaphoreType.DMA,
        ],
    )
    def sc_kernel(src_hbm, dst_hbm, dst2d_hbm, h_hbm, asv_hbm, adv_hbm, m_hbm,
                  zeros_hbm, out_hbm,
                  src_f, dst_f, dst2d, asv_t, adv_t, m_t, eexp, rows, scaled,
                  acc, sem):
        c = lax.axis_index("c")
        s = lax.axis_index("s")
        wid = c * 16 + s
        base = pl.multiple_of(wid * NE_T, 8)

        pltpu.sync_copy(zeros_hbm, scaled)
        for i in range(RPT // K):
            pltpu.sync_copy(scaled, acc.at[pl.ds(s * RPT + i * K, K)])

        pltpu.sync_copy(src_hbm.at[pl.ds(base, NE_T)], src_f)
        pltpu.sync_copy(dst_hbm.at[pl.ds(base, NE_T)], dst_f)
        pltpu.sync_copy(dst2d_hbm.at[pl.ds(wid * NCH, NCH)], dst2d)
        pltpu.sync_copy(asv_hbm, asv_t)
        pltpu.sync_copy(adv_hbm, adv_t)
        pltpu.sync_copy(m_hbm, m_t)

        def sbody(i, carry):
            o = pl.multiple_of(i * 16, 8)
            si = src_f[pl.ds(o, 16)]
            di = dst_f[pl.ds(o, 16)]
            a_s = plsc.load_gather(asv_t, [si])
            a_d = plsc.load_gather(adv_t, [di])
            m = plsc.load_gather(m_t, [di])
            t = a_s + a_d
            eexp[pl.ds(o, 16)] = jnp.exp(jnp.where(t >= 0, t, NEG * t) - m)
            return carry

        lax.fori_loop(0, NE_T // 16, sbody, 0)

        plsc.subcore_barrier()

        onehot0 = (lax.iota(jnp.int32, 16) == 0).astype(jnp.float32)

        def cbody(ch, carry):
            ebase = ch * K
            pltpu.async_copy(
                h_hbm.at[src_f.at[pl.ds(pl.multiple_of(ebase, 8), K)]],
                rows, sem).wait()

            def ebody(j, carry2):
                wv = plsc.load_gather(
                    eexp, [jnp.full((16,), ebase + j, jnp.int32)])
                for cc in range(w16):
                    scaled[j, pl.ds(cc * 16, 16)] = (
                        rows[j, pl.ds(cc * 16, 16)] * wv)
                scaled[j, pl.ds(w, 16)] = wv * onehot0
                return carry2

            lax.fori_loop(0, K, ebody, 0)
            pltpu.sync_copy(scaled, acc.at[dst2d.at[ch]], add=True)
            return carry

        lax.fori_loop(0, NCH, cbody, 0)

        plsc.subcore_barrier()
        for i in range(RPT // K):
            r0 = s * RPT + i * K
            pltpu.sync_copy(acc.at[pl.ds(r0, K)], scaled)
            pltpu.sync_copy(scaled, out_hbm.at[c, pl.ds(r0, K)])

    return sc_kernel


_sc_layer_128 = _make_sc_layer(128)
_sc_layer_48 = _make_sc_layer(48)


def kernel(x, edge_index, W1, a_src1, a_dst1, W2, a_src2, a_dst2,
           W3, a_src3, a_dst3):
    src = edge_index[0].astype(jnp.int32)
    dst = edge_index[1].astype(jnp.int32)
    dst2d = dst.reshape(NEDGES // K, K)
    z144 = jnp.zeros((K, 144), jnp.float32)
    z64 = jnp.zeros((K, 64), jnp.float32)

    h1, asv1, adv1, M1, sw1 = _tc_head(None, 128, True)(
        x, W1, a_src1[None, :], a_dst1[None, :])
    acc1 = _sc_layer_128(src, dst, dst2d, h1, asv1[:, 0], adv1[:, 0],
                         M1[:, 0], z144)

    h2, asv2, adv2, M2, sw2 = _tc_head(128, 128, False)(
        acc1, h1, sw1, W2, a_src2[None, :], a_dst2[None, :])
    acc2 = _sc_layer_128(src, dst, dst2d, h2, asv2[:, 0], adv2[:, 0],
                         M2[:, 0], z144)

    W3p = jnp.pad(W3, ((0, 0), (0, 8)))
    a_src3p = jnp.pad(a_src3, (0, 8))
    a_dst3p = jnp.pad(a_dst3, (0, 8))
    h3, asv3, adv3, M3, sw3 = _tc_head(128, 48, False)(
        acc2, h2, sw2, W3p, a_src3p[None, :], a_dst3p[None, :])
    acc3 = _sc_layer_48(src, dst, dst2d, h3, asv3[:, 0], adv3[:, 0],
                        M3[:, 0], z64)

    return _tc_final(acc3, h3, sw3, 48, 40)


# speedup vs baseline: 9.8622x; 9.8622x over previous
import functools

import jax
import jax.numpy as jnp
from jax import lax
from jax.experimental import pallas as pl
from jax.experimental.pallas import tpu as pltpu
from jax.experimental.pallas import tpu_sc as plsc

NNODES = 10000
NEDGES = 320000
W = 128
NEG = 0.2
EPS = 1e-16

NTILES = 32
CH = 128
NCH = 80
NE_T = CH * NCH
E_PAD = NE_T * NTILES
ACC_ROWS = 10240
RPT = ACC_ROWS // 16


def _lrelu(x):
    return jnp.where(x >= 0, x, NEG * x)


_RB = 2048
_NROWB = 5


def _tc_matmul_raw(x, Wm, a_s, a_d):
    def body(x_ref, W_ref, as_ref, ad_ref, h_ref, asv_ref, adv_ref):
        h = jnp.dot(x_ref[...], W_ref[...],
                    preferred_element_type=jnp.float32)
        h_ref[...] = h
        asv_ref[...] = jnp.sum(h * as_ref[...], axis=1, keepdims=True)
        adv_ref[...] = jnp.sum(h * ad_ref[...], axis=1, keepdims=True)

    out_shape = [
        jax.ShapeDtypeStruct((NNODES, W), jnp.float32),
        jax.ShapeDtypeStruct((NNODES, 1), jnp.float32),
        jax.ShapeDtypeStruct((NNODES, 1), jnp.float32),
    ]
    return pl.pallas_call(body, out_shape=out_shape)(x, Wm, a_s, a_d)


def _tc_matmul_acc(accf, accd, hp, swp, Wm, a_s, a_d):
    def body(af_ref, adn_ref, hp_ref, swp_ref, W_ref, as_ref, ad_ref,
             h_ref, asv_ref, adv_ref):
        num = af_ref[0] + af_ref[1] + swp_ref[...] * hp_ref[...]
        den = adn_ref[0] + adn_ref[1] + swp_ref[...] + EPS
        o = jnp.maximum(num / den, 0.0)
        h = jnp.dot(o, W_ref[...], preferred_element_type=jnp.float32)
        h_ref[...] = h
        asv_ref[...] = jnp.sum(h * as_ref[...], axis=1, keepdims=True)
        adv_ref[...] = jnp.sum(h * ad_ref[...], axis=1, keepdims=True)

    out_shape = [
        jax.ShapeDtypeStruct((NNODES, W), jnp.float32),
        jax.ShapeDtypeStruct((NNODES, 1), jnp.float32),
        jax.ShapeDtypeStruct((NNODES, 1), jnp.float32),
    ]
    grid = (_NROWB,)
    in_specs = [
        pl.BlockSpec((2, _RB, W), lambda i: (0, i, 0)),
        pl.BlockSpec((2, _RB, 1), lambda i: (0, i, 0)),
        pl.BlockSpec((_RB, W), lambda i: (i, 0)),
        pl.BlockSpec((_RB, 1), lambda i: (i, 0)),
        pl.BlockSpec((W, W), lambda i: (0, 0)),
        pl.BlockSpec((1, W), lambda i: (0, 0)),
        pl.BlockSpec((1, W), lambda i: (0, 0)),
    ]
    out_specs = [
        pl.BlockSpec((_RB, W), lambda i: (i, 0)),
        pl.BlockSpec((_RB, 1), lambda i: (i, 0)),
        pl.BlockSpec((_RB, 1), lambda i: (i, 0)),
    ]
    return pl.pallas_call(body, out_shape=out_shape, grid=grid,
                          in_specs=in_specs, out_specs=out_specs)(
        accf, accd, hp, swp, Wm, a_s, a_d)


def _tc_attn(asv, adv):
    def body(asv_ref, adv_ref, A_ref, sw_ref):
        A = jnp.max(asv_ref[...])
        M = _lrelu(A + adv_ref[...])
        A_ref[...] = jnp.full((1, 16), A, jnp.float32)
        sw_ref[...] = jnp.exp(_lrelu(asv_ref[...] + adv_ref[...]) - M)

    out_shape = [
        jax.ShapeDtypeStruct((1, 16), jnp.float32),
        jax.ShapeDtypeStruct((NNODES, 1), jnp.float32),
    ]
    return pl.pallas_call(body, out_shape=out_shape)(asv, adv)


def _tc_final(accf, accd, h3, sw3, cout):
    def body(af_ref, ad_ref, h_ref, sw_ref, out_ref):
        num = af_ref[0] + af_ref[1] + sw_ref[...] * h_ref[...]
        den = ad_ref[0] + ad_ref[1] + sw_ref[...] + EPS
        out_ref[...] = (num / den)[:, :cout]

    grid = (_NROWB,)
    in_specs = [
        pl.BlockSpec((2, _RB, W), lambda i: (0, i, 0)),
        pl.BlockSpec((2, _RB, 1), lambda i: (0, i, 0)),
        pl.BlockSpec((_RB, W), lambda i: (i, 0)),
        pl.BlockSpec((_RB, 1), lambda i: (i, 0)),
    ]
    out_specs = pl.BlockSpec((_RB, cout), lambda i: (i, 0))
    return pl.pallas_call(
        body, out_shape=jax.ShapeDtypeStruct((NNODES, cout), jnp.float32),
        grid=grid, in_specs=in_specs, out_specs=out_specs,
    )(accf, accd, h3, sw3)


_mesh = plsc.VectorSubcoreMesh(core_axis_name="c", subcore_axis_name="s",
                               num_cores=2, num_subcores=16)


@functools.partial(
    pl.kernel,
    out_type=[
        jax.ShapeDtypeStruct((2, ACC_ROWS, W), jnp.float32),
        jax.ShapeDtypeStruct((2, ACC_ROWS), jnp.float32),
    ],
    mesh=_mesh,
    compiler_params=pltpu.CompilerParams(needs_layout_passes=False),
    scratch_types=[
        pltpu.VMEM((CH,), jnp.int32),
        pltpu.VMEM((CH,), jnp.int32),
        pltpu.VMEM((ACC_ROWS,), jnp.float32),
        pltpu.VMEM((ACC_ROWS,), jnp.float32),
        pltpu.VMEM((16,), jnp.float32),
        pltpu.VMEM((CH,), jnp.float32),
        pltpu.VMEM((CH, W), jnp.float32),
        pltpu.VMEM((RPT,), jnp.float32),
        pltpu.VMEM_SHARED((ACC_ROWS, W), jnp.float32),
        pltpu.VMEM_SHARED((ACC_ROWS,), jnp.float32),
        pltpu.SemaphoreType.DMA,
        pltpu.SemaphoreType.DMA,
        pltpu.SemaphoreType.DMA,
    ],
)
def _sc_edge_phase(src_hbm, dst_hbm, h_hbm, asv_hbm, adv_hbm, a_hbm,
                   zeros_hbm, outf_hbm, outd_hbm,
                   s_ch, d_ch, asv_t, adv_t, a_t, ebuf, rows, dbuf,
                   accf, accd, semg, sems, semd):
    c = lax.axis_index("c")
    s = lax.axis_index("s")
    wid = c * 16 + s
    base = pl.multiple_of(wid * NE_T, 8)
    row0 = pl.multiple_of(s * RPT, 8)

    pltpu.sync_copy(zeros_hbm, rows)
    for i in range(RPT // CH):
        pltpu.sync_copy(rows, accf.at[pl.ds(row0 + i * CH, CH)])
    for i in range(RPT // 16):
        dbuf[pl.ds(i * 16, 16)] = jnp.zeros((16,), jnp.float32)
    pltpu.sync_copy(dbuf, accd.at[pl.ds(row0, RPT)])

    pltpu.sync_copy(asv_hbm, asv_t)
    pltpu.sync_copy(adv_hbm, adv_t)
    pltpu.sync_copy(a_hbm, a_t)

    plsc.subcore_barrier()

    iota16 = lax.iota(jnp.int32, 16)
    avec = a_t[pl.ds(0, 16)]

    def chunk_body(ch, carry):
        o0 = pl.multiple_of(base + ch * CH, 8)
        pltpu.sync_copy(src_hbm.at[pl.ds(o0, CH)], s_ch)
        pltpu.sync_copy(dst_hbm.at[pl.ds(o0, CH)], d_ch)
        descs = []
        for g in range(CH // 16):
            si = s_ch[pl.ds(g * 16, 16)]
            di = d_ch[pl.ds(g * 16, 16)]
            a_s = plsc.load_gather(asv_t, [si])
            a_d = plsc.load_gather(adv_t, [di])
            t = a_s + a_d
            m = _lrelu(avec + a_d)
            ebuf[pl.ds(g * 16, 16)] = jnp.exp(_lrelu(t) - m)
            descs.append(
                pltpu.async_copy(h_hbm.at[si], rows.at[pl.ds(g * 16, 16)],
                                 semg))
        for d in descs:
            d.wait()

        def scale_body(j, carry2):
            js = jnp.full((16,), j, jnp.int32)
            wv = plsc.load_gather(ebuf, [js])
            for cc in range(W // 16):
                col = iota16 + (cc * 16)
                v = plsc.load_gather(rows, [js, col])
                plsc.store_scatter(rows, [js, col], v * wv)
            return carry2

        lax.fori_loop(0, CH, scale_body, 0)

        descs = []
        for g in range(CH // 16):
            di = d_ch[pl.ds(g * 16, 16)]
            descs.append(
                pltpu.async_copy(rows.at[pl.ds(g * 16, 16)], accf.at[di],
                                 sems, add=True))
            descs.append(
                pltpu.async_copy(ebuf.at[pl.ds(g * 16, 16)],
                                 accd.at[di], semd, add=True))
        for d in descs:
            d.wait()
        return carry

    lax.fori_loop(0, NCH, chunk_body, 0)

    plsc.subcore_barrier()
    for i in range(RPT // CH):
        r0 = pl.multiple_of(row0 + i * CH, 8)
        pltpu.sync_copy(accf.at[pl.ds(r0, CH)], rows)
        pltpu.sync_copy(rows, outf_hbm.at[c, pl.ds(r0, CH)])
    pltpu.sync_copy(accd.at[pl.ds(row0, RPT)], dbuf)
    pltpu.sync_copy(dbuf, outd_hbm.at[c, pl.ds(row0, RPT)])


def _pad_nodes(v):
    return jnp.pad(v[:, 0], (0, ACC_ROWS - NNODES))


def kernel(x, edge_index, W1, a_src1, a_dst1, W2, a_src2, a_dst2,
           W3, a_src3, a_dst3):
    src = jnp.pad(edge_index[0].astype(jnp.int32), (0, E_PAD - NEDGES))
    dst = jnp.pad(edge_index[1].astype(jnp.int32), (0, E_PAD - NEDGES),
                  constant_values=ACC_ROWS - 1)
    zeros2d = jnp.zeros((CH, W), jnp.float32)

    h1, asv1, adv1 = _tc_matmul_raw(x, W1, a_src1[None, :], a_dst1[None, :])
    A1, sw1 = _tc_attn(asv1, adv1)
    af1, ad1 = _sc_edge_phase(src, dst, h1, _pad_nodes(asv1),
                              _pad_nodes(adv1), A1[0], zeros2d)

    h2, asv2, adv2 = _tc_matmul_acc(
        af1, ad1[..., None], h1, sw1, W2, a_src2[None, :], a_dst2[None, :])
    A2, sw2 = _tc_attn(asv2, adv2)
    af2, ad2 = _sc_edge_phase(src, dst, h2, _pad_nodes(asv2),
                              _pad_nodes(adv2), A2[0], zeros2d)

    W3p = jnp.pad(W3, ((0, 0), (0, W - W3.shape[1])))
    a_src3p = jnp.pad(a_src3, (0, W - a_src3.shape[0]))
    a_dst3p = jnp.pad(a_dst3, (0, W - a_dst3.shape[0]))
    h3, asv3, adv3 = _tc_matmul_acc(
        af2, ad2[..., None], h2, sw2, W3p, a_src3p[None, :],
        a_dst3p[None, :])
    A3, sw3 = _tc_attn(asv3, adv3)
    af3, ad3 = _sc_edge_phase(src, dst, h3, _pad_nodes(asv3),
                              _pad_nodes(adv3), A3[0], zeros2d)

    return _tc_final(af3, ad3[..., None], h3, sw3, 40)


# speedup vs baseline: 14.6464x; 1.4851x over previous
import functools

import jax
import jax.numpy as jnp
from jax import lax
from jax.experimental import pallas as pl
from jax.experimental.pallas import tpu as pltpu
from jax.experimental.pallas import tpu_sc as plsc

NNODES = 10000
NEDGES = 320000
W = 128
NEG = 0.2
EPS = 1e-16

NTILES = 32
CH = 128
NCH = 80
NE_T = CH * NCH
E_PAD = NE_T * NTILES
ACC_ROWS = 10240
RPT = ACC_ROWS // 16


def _lrelu(x):
    return jnp.where(x >= 0, x, NEG * x)


_RB = 2048
_NROWB = 5


def _tc_matmul_raw(x, Wm, a_s, a_d):
    def body(x_ref, W_ref, as_ref, ad_ref, h_ref, asv_ref, adv_ref):
        h = jnp.dot(x_ref[...], W_ref[...],
                    preferred_element_type=jnp.float32)
        h_ref[...] = h
        asv_ref[...] = jnp.sum(h * as_ref[...], axis=1, keepdims=True)
        adv_ref[...] = jnp.sum(h * ad_ref[...], axis=1, keepdims=True)

    out_shape = [
        jax.ShapeDtypeStruct((NNODES, W), jnp.float32),
        jax.ShapeDtypeStruct((NNODES, 1), jnp.float32),
        jax.ShapeDtypeStruct((NNODES, 1), jnp.float32),
    ]
    return pl.pallas_call(body, out_shape=out_shape)(x, Wm, a_s, a_d)


def _tc_matmul_acc(accf, accd, hp, swp, Wm, a_s, a_d):
    def body(af_ref, adn_ref, hp_ref, swp_ref, W_ref, as_ref, ad_ref,
             h_ref, asv_ref, adv_ref):
        num = af_ref[0] + af_ref[1] + swp_ref[...] * hp_ref[...]
        den = adn_ref[0] + adn_ref[1] + swp_ref[...] + EPS
        o = jnp.maximum(num / den, 0.0)
        h = jnp.dot(o, W_ref[...], preferred_element_type=jnp.float32)
        h_ref[...] = h
        asv_ref[...] = jnp.sum(h * as_ref[...], axis=1, keepdims=True)
        adv_ref[...] = jnp.sum(h * ad_ref[...], axis=1, keepdims=True)

    out_shape = [
        jax.ShapeDtypeStruct((NNODES, W), jnp.float32),
        jax.ShapeDtypeStruct((NNODES, 1), jnp.float32),
        jax.ShapeDtypeStruct((NNODES, 1), jnp.float32),
    ]
    grid = (_NROWB,)
    in_specs = [
        pl.BlockSpec((2, _RB, W), lambda i: (0, i, 0)),
        pl.BlockSpec((2, _RB, 1), lambda i: (0, i, 0)),
        pl.BlockSpec((_RB, W), lambda i: (i, 0)),
        pl.BlockSpec((_RB, 1), lambda i: (i, 0)),
        pl.BlockSpec((W, W), lambda i: (0, 0)),
        pl.BlockSpec((1, W), lambda i: (0, 0)),
        pl.BlockSpec((1, W), lambda i: (0, 0)),
    ]
    out_specs = [
        pl.BlockSpec((_RB, W), lambda i: (i, 0)),
        pl.BlockSpec((_RB, 1), lambda i: (i, 0)),
        pl.BlockSpec((_RB, 1), lambda i: (i, 0)),
    ]
    return pl.pallas_call(body, out_shape=out_shape, grid=grid,
                          in_specs=in_specs, out_specs=out_specs)(
        accf, accd, hp, swp, Wm, a_s, a_d)


def _tc_attn(asv, adv):
    def body(asv_ref, adv_ref, A_ref, sw_ref):
        A = jnp.max(asv_ref[...])
        M = _lrelu(A + adv_ref[...])
        A_ref[...] = jnp.full((1, 16), A, jnp.float32)
        sw_ref[...] = jnp.exp(_lrelu(asv_ref[...] + adv_ref[...]) - M)

    out_shape = [
        jax.ShapeDtypeStruct((1, 16), jnp.float32),
        jax.ShapeDtypeStruct((NNODES, 1), jnp.float32),
    ]
    return pl.pallas_call(body, out_shape=out_shape)(asv, adv)


def _tc_final(accf, accd, h3, sw3, cout):
    def body(af_ref, ad_ref, h_ref, sw_ref, out_ref):
        num = af_ref[0] + af_ref[1] + sw_ref[...] * h_ref[...]
        den = ad_ref[0] + ad_ref[1] + sw_ref[...] + EPS
        out_ref[...] = (num / den)[:, :cout]

    grid = (_NROWB,)
    in_specs = [
        pl.BlockSpec((2, _RB, W), lambda i: (0, i, 0)),
        pl.BlockSpec((2, _RB, 1), lambda i: (0, i, 0)),
        pl.BlockSpec((_RB, W), lambda i: (i, 0)),
        pl.BlockSpec((_RB, 1), lambda i: (i, 0)),
    ]
    out_specs = pl.BlockSpec((_RB, cout), lambda i: (i, 0))
    return pl.pallas_call(
        body, out_shape=jax.ShapeDtypeStruct((NNODES, cout), jnp.float32),
        grid=grid, in_specs=in_specs, out_specs=out_specs,
    )(accf, accd, h3, sw3)


_mesh = plsc.VectorSubcoreMesh(core_axis_name="c", subcore_axis_name="s",
                               num_cores=2, num_subcores=16)


@functools.partial(
    pl.kernel,
    out_type=[
        jax.ShapeDtypeStruct((2, ACC_ROWS, W), jnp.float32),
        jax.ShapeDtypeStruct((2, ACC_ROWS), jnp.float32),
    ],
    mesh=_mesh,
    compiler_params=pltpu.CompilerParams(needs_layout_passes=False),
    scratch_types=[
        pltpu.VMEM((CH,), jnp.int32),
        pltpu.VMEM((CH,), jnp.int32),
        pltpu.VMEM((ACC_ROWS,), jnp.float32),
        pltpu.VMEM((ACC_ROWS,), jnp.float32),
        pltpu.VMEM((16,), jnp.float32),
        pltpu.VMEM((CH,), jnp.float32),
        pltpu.VMEM((CH, W), jnp.float32),
        pltpu.VMEM((RPT,), jnp.float32),
        pltpu.VMEM_SHARED((ACC_ROWS, W), jnp.float32),
        pltpu.VMEM_SHARED((ACC_ROWS,), jnp.float32),
        pltpu.SemaphoreType.DMA,
        pltpu.SemaphoreType.DMA,
        pltpu.SemaphoreType.DMA,
    ],
)
def _sc_edge_phase(src_hbm, dst_hbm, h_hbm, asv_hbm, adv_hbm, a_hbm,
                   zeros_hbm, outf_hbm, outd_hbm,
                   s_ch, d_ch, asv_t, adv_t, a_t, ebuf, rows, dbuf,
                   accf, accd, semg, sems, semd):
    c = lax.axis_index("c")
    s = lax.axis_index("s")
    wid = c * 16 + s
    base = pl.multiple_of(wid * NE_T, 8)
    row0 = pl.multiple_of(s * RPT, 8)

    pltpu.sync_copy(zeros_hbm, rows)
    for i in range(RPT // CH):
        pltpu.sync_copy(rows, accf.at[pl.ds(row0 + i * CH, CH)])
    for i in range(RPT // 16):
        dbuf[pl.ds(i * 16, 16)] = jnp.zeros((16,), jnp.float32)
    pltpu.sync_copy(dbuf, accd.at[pl.ds(row0, RPT)])

    pltpu.sync_copy(asv_hbm, asv_t)
    pltpu.sync_copy(adv_hbm, adv_t)
    pltpu.sync_copy(a_hbm, a_t)

    plsc.subcore_barrier()

    iota16 = lax.iota(jnp.int32, 16)
    avec = a_t[pl.ds(0, 16)]

    def chunk_body(ch, carry):
        o0 = pl.multiple_of(base + ch * CH, 8)
        pltpu.sync_copy(src_hbm.at[pl.ds(o0, CH)], s_ch)
        pltpu.sync_copy(dst_hbm.at[pl.ds(o0, CH)], d_ch)
        gdesc = pltpu.async_copy(h_hbm.at[s_ch], rows, semg)
        for g in range(CH // 16):
            si = s_ch[pl.ds(g * 16, 16)]
            di = d_ch[pl.ds(g * 16, 16)]
            a_s = plsc.load_gather(asv_t, [si])
            a_d = plsc.load_gather(adv_t, [di])
            t = a_s + a_d
            m = _lrelu(avec + a_d)
            ebuf[pl.ds(g * 16, 16)] = jnp.exp(_lrelu(t) - m)
        gdesc.wait()

        def scale_body(j):
            js = jnp.full((16,), j, jnp.int32)
            wv = plsc.load_gather(ebuf, [js])
            for cc in range(W // 16):
                col = iota16 + (cc * 16)
                v = plsc.load_gather(rows, [js, col])
                plsc.store_scatter(rows, [js, col], v * wv)

        plsc.parallel_loop(0, CH, 1, unroll=4, carry=None)(scale_body)

        d1 = pltpu.async_copy(rows, accf.at[d_ch], sems, add=True)
        d2 = pltpu.async_copy(ebuf, accd.at[d_ch], semd, add=True)
        d1.wait()
        d2.wait()
        return carry

    lax.fori_loop(0, NCH, chunk_body, 0)

    plsc.subcore_barrier()
    for i in range(RPT // CH):
        r0 = pl.multiple_of(row0 + i * CH, 8)
        pltpu.sync_copy(accf.at[pl.ds(r0, CH)], rows)
        pltpu.sync_copy(rows, outf_hbm.at[c, pl.ds(r0, CH)])
    pltpu.sync_copy(accd.at[pl.ds(row0, RPT)], dbuf)
    pltpu.sync_copy(dbuf, outd_hbm.at[c, pl.ds(row0, RPT)])


def _pad_nodes(v):
    return jnp.pad(v[:, 0], (0, ACC_ROWS - NNODES))


def kernel(x, edge_index, W1, a_src1, a_dst1, W2, a_src2, a_dst2,
           W3, a_src3, a_dst3):
    src = jnp.pad(edge_index[0].astype(jnp.int32), (0, E_PAD - NEDGES))
    dst = jnp.pad(edge_index[1].astype(jnp.int32), (0, E_PAD - NEDGES),
                  constant_values=ACC_ROWS - 1)
    zeros2d = jnp.zeros((CH, W), jnp.float32)

    h1, asv1, adv1 = _tc_matmul_raw(x, W1, a_src1[None, :], a_dst1[None, :])
    A1, sw1 = _tc_attn(asv1, adv1)
    af1, ad1 = _sc_edge_phase(src, dst, h1, _pad_nodes(asv1),
                              _pad_nodes(adv1), A1[0], zeros2d)

    h2, asv2, adv2 = _tc_matmul_acc(
        af1, ad1[..., None], h1, sw1, W2, a_src2[None, :], a_dst2[None, :])
    A2, sw2 = _tc_attn(asv2, adv2)
    af2, ad2 = _sc_edge_phase(src, dst, h2, _pad_nodes(asv2),
                              _pad_nodes(adv2), A2[0], zeros2d)

    W3p = jnp.pad(W3, ((0, 0), (0, W - W3.shape[1])))
    a_src3p = jnp.pad(a_src3, (0, W - a_src3.shape[0]))
    a_dst3p = jnp.pad(a_dst3, (0, W - a_dst3.shape[0]))
    h3, asv3, adv3 = _tc_matmul_acc(
        af2, ad2[..., None], h2, sw2, W3p, a_src3p[None, :],
        a_dst3p[None, :])
    A3, sw3 = _tc_attn(asv3, adv3)
    af3, ad3 = _sc_edge_phase(src, dst, h3, _pad_nodes(asv3),
                              _pad_nodes(adv3), A3[0], zeros2d)

    return _tc_final(af3, ad3[..., None], h3, sw3, 40)


# speedup vs baseline: 15.5705x; 1.0631x over previous
import functools

import jax
import jax.numpy as jnp
from jax import lax
from jax.experimental import pallas as pl
from jax.experimental.pallas import tpu as pltpu
from jax.experimental.pallas import tpu_sc as plsc

NNODES = 10000
NEDGES = 320000
W = 128
NEG = 0.2
EPS = 1e-16

NTILES = 32
CH = 128
NCH = 80
NE_T = CH * NCH
E_PAD = NE_T * NTILES
ACC_ROWS = 10240
RPT = ACC_ROWS // 16


def _lrelu(x):
    return jnp.where(x >= 0, x, NEG * x)


_RB = 2048
_NROWB = 5


def _tc_matmul_raw(x, Wm, a_s, a_d):
    def body(x_ref, W_ref, as_ref, ad_ref, h_ref, asv_ref, adv_ref):
        h = jnp.dot(x_ref[...], W_ref[...],
                    preferred_element_type=jnp.float32)
        h_ref[...] = h
        asv_ref[...] = jnp.sum(h * as_ref[...], axis=1, keepdims=True)
        adv_ref[...] = jnp.sum(h * ad_ref[...], axis=1, keepdims=True)

    out_shape = [
        jax.ShapeDtypeStruct((NNODES, W), jnp.float32),
        jax.ShapeDtypeStruct((NNODES, 1), jnp.float32),
        jax.ShapeDtypeStruct((NNODES, 1), jnp.float32),
    ]
    return pl.pallas_call(body, out_shape=out_shape)(x, Wm, a_s, a_d)


def _tc_matmul_acc(accf, accd, hp, swp, Wm, a_s, a_d):
    def body(af_ref, adn_ref, hp_ref, swp_ref, W_ref, as_ref, ad_ref,
             h_ref, asv_ref, adv_ref):
        num = af_ref[0] + af_ref[1] + swp_ref[...] * hp_ref[...]
        den = adn_ref[0] + adn_ref[1] + swp_ref[...] + EPS
        o = jnp.maximum(num / den, 0.0)
        h = jnp.dot(o, W_ref[...], preferred_element_type=jnp.float32)
        h_ref[...] = h
        asv_ref[...] = jnp.sum(h * as_ref[...], axis=1, keepdims=True)
        adv_ref[...] = jnp.sum(h * ad_ref[...], axis=1, keepdims=True)

    out_shape = [
        jax.ShapeDtypeStruct((NNODES, W), jnp.float32),
        jax.ShapeDtypeStruct((NNODES, 1), jnp.float32),
        jax.ShapeDtypeStruct((NNODES, 1), jnp.float32),
    ]
    grid = (_NROWB,)
    in_specs = [
        pl.BlockSpec((2, _RB, W), lambda i: (0, i, 0)),
        pl.BlockSpec((2, _RB, 1), lambda i: (0, i, 0)),
        pl.BlockSpec((_RB, W), lambda i: (i, 0)),
        pl.BlockSpec((_RB, 1), lambda i: (i, 0)),
        pl.BlockSpec((W, W), lambda i: (0, 0)),
        pl.BlockSpec((1, W), lambda i: (0, 0)),
        pl.BlockSpec((1, W), lambda i: (0, 0)),
    ]
    out_specs = [
        pl.BlockSpec((_RB, W), lambda i: (i, 0)),
        pl.BlockSpec((_RB, 1), lambda i: (i, 0)),
        pl.BlockSpec((_RB, 1), lambda i: (i, 0)),
    ]
    return pl.pallas_call(body, out_shape=out_shape, grid=grid,
                          in_specs=in_specs, out_specs=out_specs)(
        accf, accd, hp, swp, Wm, a_s, a_d)


def _tc_attn(asv, adv):
    def body(asv_ref, adv_ref, A_ref, sw_ref):
        A = jnp.max(asv_ref[...])
        M = _lrelu(A + adv_ref[...])
        A_ref[...] = jnp.full((1, 16), A, jnp.float32)
        sw_ref[...] = jnp.exp(_lrelu(asv_ref[...] + adv_ref[...]) - M)

    out_shape = [
        jax.ShapeDtypeStruct((1, 16), jnp.float32),
        jax.ShapeDtypeStruct((NNODES, 1), jnp.float32),
    ]
    return pl.pallas_call(body, out_shape=out_shape)(asv, adv)


def _tc_final(accf, accd, h3, sw3, cout):
    def body(af_ref, ad_ref, h_ref, sw_ref, out_ref):
        num = af_ref[0] + af_ref[1] + sw_ref[...] * h_ref[...]
        den = ad_ref[0] + ad_ref[1] + sw_ref[...] + EPS
        out_ref[...] = (num / den)[:, :cout]

    grid = (_NROWB,)
    in_specs = [
        pl.BlockSpec((2, _RB, W), lambda i: (0, i, 0)),
        pl.BlockSpec((2, _RB, 1), lambda i: (0, i, 0)),
        pl.BlockSpec((_RB, W), lambda i: (i, 0)),
        pl.BlockSpec((_RB, 1), lambda i: (i, 0)),
    ]
    out_specs = pl.BlockSpec((_RB, cout), lambda i: (i, 0))
    return pl.pallas_call(
        body, out_shape=jax.ShapeDtypeStruct((NNODES, cout), jnp.float32),
        grid=grid, in_specs=in_specs, out_specs=out_specs,
    )(accf, accd, h3, sw3)


_mesh = plsc.VectorSubcoreMesh(core_axis_name="c", subcore_axis_name="s",
                               num_cores=2, num_subcores=16)


@functools.partial(
    pl.kernel,
    out_type=jax.ShapeDtypeStruct((E_PAD,), jnp.float32),
    mesh=_mesh,
    compiler_params=pltpu.CompilerParams(needs_layout_passes=False),
    scratch_types=[
        pltpu.VMEM((CH,), jnp.int32),
        pltpu.VMEM((CH,), jnp.int32),
        pltpu.VMEM((ACC_ROWS,), jnp.float32),
        pltpu.VMEM((ACC_ROWS,), jnp.float32),
        pltpu.VMEM((16,), jnp.float32),
        pltpu.VMEM((CH,), jnp.float32),
    ],
)
def _sc_weights(src_hbm, dst_hbm, asv_hbm, adv_hbm, a_hbm, w_hbm,
                s_ch, d_ch, asv_t, adv_t, a_t, w_ch):
    c = lax.axis_index("c")
    s = lax.axis_index("s")
    wid = c * 16 + s
    base = pl.multiple_of(wid * NE_T, 8)

    pltpu.sync_copy(asv_hbm, asv_t)
    pltpu.sync_copy(adv_hbm, adv_t)
    pltpu.sync_copy(a_hbm, a_t)
    avec = a_t[pl.ds(0, 16)]

    def chunk_body(ch, carry):
        o0 = pl.multiple_of(base + ch * CH, 8)
        pltpu.sync_copy(src_hbm.at[pl.ds(o0, CH)], s_ch)
        pltpu.sync_copy(dst_hbm.at[pl.ds(o0, CH)], d_ch)
        for g in range(CH // 16):
            si = s_ch[pl.ds(g * 16, 16)]
            di = d_ch[pl.ds(g * 16, 16)]
            a_s = plsc.load_gather(asv_t, [si])
            a_d = plsc.load_gather(adv_t, [di])
            t = a_s + a_d
            m = _lrelu(avec + a_d)
            w_ch[pl.ds(g * 16, 16)] = jnp.exp(_lrelu(t) - m)
        pltpu.sync_copy(w_ch, w_hbm.at[pl.ds(o0, CH)])
        return carry

    lax.fori_loop(0, NCH, chunk_body, 0)


@functools.partial(
    pl.kernel,
    out_type=[
        jax.ShapeDtypeStruct((2, ACC_ROWS, W), jnp.float32),
        jax.ShapeDtypeStruct((2, ACC_ROWS), jnp.float32),
    ],
    mesh=_mesh,
    compiler_params=pltpu.CompilerParams(needs_layout_passes=False),
    scratch_types=[
        pltpu.VMEM((CH,), jnp.int32),
        pltpu.VMEM((CH,), jnp.int32),
        pltpu.VMEM((CH,), jnp.int32),
        pltpu.VMEM((CH,), jnp.int32),
        pltpu.VMEM((CH,), jnp.float32),
        pltpu.VMEM((CH,), jnp.float32),
        pltpu.VMEM((CH, W), jnp.float32),
        pltpu.VMEM((CH, W), jnp.float32),
        pltpu.VMEM((RPT,), jnp.float32),
        pltpu.VMEM_SHARED((ACC_ROWS, W), jnp.float32),
        pltpu.VMEM_SHARED((ACC_ROWS,), jnp.float32),
        pltpu.SemaphoreType.DMA,
        pltpu.SemaphoreType.DMA,
        pltpu.SemaphoreType.DMA,
        pltpu.SemaphoreType.DMA,
    ],
)
def _sc_edge_phase(src_hbm, dst_hbm, w_hbm, h_hbm, zeros_hbm,
                   outf_hbm, outd_hbm,
                   s_a, d_a, s_b, d_b, eb_a, eb_b, rows_a, rows_b, dbuf,
                   accf, accd, semg_a, semg_b, sems_a, sems_b):
    c = lax.axis_index("c")
    s = lax.axis_index("s")
    wid = c * 16 + s
    base = pl.multiple_of(wid * NE_T, 8)
    row0 = pl.multiple_of(s * RPT, 8)

    pltpu.sync_copy(zeros_hbm, rows_a)
    for i in range(RPT // CH):
        pltpu.sync_copy(rows_a, accf.at[pl.ds(row0 + i * CH, CH)])
    for i in range(RPT // 16):
        dbuf[pl.ds(i * 16, 16)] = jnp.zeros((16,), jnp.float32)
    pltpu.sync_copy(dbuf, accd.at[pl.ds(row0, RPT)])

    plsc.subcore_barrier()

    iota16 = lax.iota(jnp.int32, 16)

    def prep(cidx, sx, dx, ebx, rowsx, semgx):
        o0 = pl.multiple_of(base + cidx * CH, 8)
        pltpu.sync_copy(src_hbm.at[pl.ds(o0, CH)], sx)
        pltpu.sync_copy(dst_hbm.at[pl.ds(o0, CH)], dx)
        pltpu.sync_copy(w_hbm.at[pl.ds(o0, CH)], ebx)
        pltpu.async_copy(h_hbm.at[sx], rowsx, semgx)

    def process(sx, dx, ebx, rowsx, semgx, semsx):
        pltpu.make_async_copy(h_hbm.at[sx], rowsx, semgx).wait()

        def scale_body(j):
            js = jnp.full((16,), j, jnp.int32)
            wv = plsc.load_gather(ebx, [js])
            for cc in range(W // 16):
                col = iota16 + (cc * 16)
                v = plsc.load_gather(rowsx, [js, col])
                plsc.store_scatter(rowsx, [js, col], v * wv)

        plsc.parallel_loop(0, CH, 1, unroll=4, carry=None)(scale_body)
        pltpu.async_copy(rowsx, accf.at[dx], semsx, add=True)
        pltpu.async_copy(ebx, accd.at[dx], semsx, add=True)

    def wait_scatter(dx, ebx, rowsx, semsx):
        pltpu.make_async_copy(rowsx, accf.at[dx], semsx).wait()
        pltpu.make_async_copy(ebx, accd.at[dx], semsx).wait()

    prep(0, s_a, d_a, eb_a, rows_a, semg_a)
    prep(1, s_b, d_b, eb_b, rows_b, semg_b)
    npairs = NCH // 2

    def pair_body(k2, carry):
        cidx = k2 * 2
        process(s_a, d_a, eb_a, rows_a, semg_a, sems_a)
        process(s_b, d_b, eb_b, rows_b, semg_b, sems_b)

        @pl.when(k2 + 1 < npairs)
        def _():
            wait_scatter(d_a, eb_a, rows_a, sems_a)
            prep(cidx + 2, s_a, d_a, eb_a, rows_a, semg_a)
            wait_scatter(d_b, eb_b, rows_b, sems_b)
            prep(cidx + 3, s_b, d_b, eb_b, rows_b, semg_b)

        return carry

    lax.fori_loop(0, npairs, pair_body, 0)
    wait_scatter(d_a, eb_a, rows_a, sems_a)
    wait_scatter(d_b, eb_b, rows_b, sems_b)

    plsc.subcore_barrier()
    for i in range(RPT // CH):
        r0 = pl.multiple_of(row0 + i * CH, 8)
        pltpu.sync_copy(accf.at[pl.ds(r0, CH)], rows_a)
        pltpu.sync_copy(rows_a, outf_hbm.at[c, pl.ds(r0, CH)])
    pltpu.sync_copy(accd.at[pl.ds(row0, RPT)], dbuf)
    pltpu.sync_copy(dbuf, outd_hbm.at[c, pl.ds(row0, RPT)])


def _pad_nodes(v):
    return jnp.pad(v[:, 0], (0, ACC_ROWS - NNODES))


def kernel(x, edge_index, W1, a_src1, a_dst1, W2, a_src2, a_dst2,
           W3, a_src3, a_dst3):
    src = jnp.pad(edge_index[0].astype(jnp.int32), (0, E_PAD - NEDGES))
    dst = jnp.pad(edge_index[1].astype(jnp.int32), (0, E_PAD - NEDGES),
                  constant_values=ACC_ROWS - 1)
    zeros2d = jnp.zeros((CH, W), jnp.float32)

    h1, asv1, adv1 = _tc_matmul_raw(x, W1, a_src1[None, :], a_dst1[None, :])
    A1, sw1 = _tc_attn(asv1, adv1)
    w1 = _sc_weights(src, dst, _pad_nodes(asv1), _pad_nodes(adv1), A1[0])
    af1, ad1 = _sc_edge_phase(src, dst, w1, h1, zeros2d)

    h2, asv2, adv2 = _tc_matmul_acc(
        af1, ad1[..., None], h1, sw1, W2, a_src2[None, :], a_dst2[None, :])
    A2, sw2 = _tc_attn(asv2, adv2)
    w2 = _sc_weights(src, dst, _pad_nodes(asv2), _pad_nodes(adv2), A2[0])
    af2, ad2 = _sc_edge_phase(src, dst, w2, h2, zeros2d)

    W3p = jnp.pad(W3, ((0, 0), (0, W - W3.shape[1])))
    a_src3p = jnp.pad(a_src3, (0, W - a_src3.shape[0]))
    a_dst3p = jnp.pad(a_dst3, (0, W - a_dst3.shape[0]))
    h3, asv3, adv3 = _tc_matmul_acc(
        af2, ad2[..., None], h2, sw2, W3p, a_src3p[None, :],
        a_dst3p[None, :])
    A3, sw3 = _tc_attn(asv3, adv3)
    w3 = _sc_weights(src, dst, _pad_nodes(asv3), _pad_nodes(adv3), A3[0])
    af3, ad3 = _sc_edge_phase(src, dst, w3, h3, zeros2d)

    return _tc_final(af3, ad3[..., None], h3, sw3, 40)


# speedup vs baseline: 15.8594x; 1.0186x over previous
import functools

import jax
import jax.numpy as jnp
from jax import lax
from jax.experimental import pallas as pl
from jax.experimental.pallas import tpu as pltpu
from jax.experimental.pallas import tpu_sc as plsc

NNODES = 10000
NEDGES = 320000
W = 128
NEG = 0.2
EPS = 1e-16

NTILES = 32
CH = 80
NCH = 128
NE_T = CH * NCH
E_PAD = NE_T * NTILES
ACC_ROWS = 10240
RPT = ACC_ROWS // 16


def _lrelu(x):
    return jnp.where(x >= 0, x, NEG * x)


_RB = 2048
_NROWB = 5


def _tc_matmul_raw(x, Wm, a_s, a_d):
    def body(x_ref, W_ref, as_ref, ad_ref, h_ref, asv_ref, adv_ref):
        h = jnp.dot(x_ref[...], W_ref[...],
                    preferred_element_type=jnp.float32)
        h_ref[...] = h
        asv_ref[...] = jnp.sum(h * as_ref[...], axis=1, keepdims=True)
        adv_ref[...] = jnp.sum(h * ad_ref[...], axis=1, keepdims=True)

    out_shape = [
        jax.ShapeDtypeStruct((NNODES, W), jnp.float32),
        jax.ShapeDtypeStruct((NNODES, 1), jnp.float32),
        jax.ShapeDtypeStruct((NNODES, 1), jnp.float32),
    ]
    return pl.pallas_call(body, out_shape=out_shape)(x, Wm, a_s, a_d)


def _tc_matmul_acc(accf, accd, hp, swp, Wm, a_s, a_d):
    def body(af_ref, adn_ref, hp_ref, swp_ref, W_ref, as_ref, ad_ref,
             h_ref, asv_ref, adv_ref):
        num = af_ref[0] + af_ref[1] + swp_ref[...] * hp_ref[...]
        den = adn_ref[0] + adn_ref[1] + swp_ref[...] + EPS
        o = jnp.maximum(num / den, 0.0)
        h = jnp.dot(o, W_ref[...], preferred_element_type=jnp.float32)
        h_ref[...] = h
        asv_ref[...] = jnp.sum(h * as_ref[...], axis=1, keepdims=True)
        adv_ref[...] = jnp.sum(h * ad_ref[...], axis=1, keepdims=True)

    out_shape = [
        jax.ShapeDtypeStruct((NNODES, W), jnp.float32),
        jax.ShapeDtypeStruct((NNODES, 1), jnp.float32),
        jax.ShapeDtypeStruct((NNODES, 1), jnp.float32),
    ]
    grid = (_NROWB,)
    in_specs = [
        pl.BlockSpec((2, _RB, W), lambda i: (0, i, 0)),
        pl.BlockSpec((2, _RB, 1), lambda i: (0, i, 0)),
        pl.BlockSpec((_RB, W), lambda i: (i, 0)),
        pl.BlockSpec((_RB, 1), lambda i: (i, 0)),
        pl.BlockSpec((W, W), lambda i: (0, 0)),
        pl.BlockSpec((1, W), lambda i: (0, 0)),
        pl.BlockSpec((1, W), lambda i: (0, 0)),
    ]
    out_specs = [
        pl.BlockSpec((_RB, W), lambda i: (i, 0)),
        pl.BlockSpec((_RB, 1), lambda i: (i, 0)),
        pl.BlockSpec((_RB, 1), lambda i: (i, 0)),
    ]
    return pl.pallas_call(body, out_shape=out_shape, grid=grid,
                          in_specs=in_specs, out_specs=out_specs)(
        accf, accd, hp, swp, Wm, a_s, a_d)


def _tc_attn(asv, adv):
    def body(asv_ref, adv_ref, A_ref, sw_ref):
        A = jnp.max(asv_ref[...])
        M = _lrelu(A + adv_ref[...])
        A_ref[...] = jnp.full((1, 16), A, jnp.float32)
        sw_ref[...] = jnp.exp(_lrelu(asv_ref[...] + adv_ref[...]) - M)

    out_shape = [
        jax.ShapeDtypeStruct((1, 16), jnp.float32),
        jax.ShapeDtypeStruct((NNODES, 1), jnp.float32),
    ]
    return pl.pallas_call(body, out_shape=out_shape)(asv, adv)


def _tc_final(accf, accd, h3, sw3, cout):
    def body(af_ref, ad_ref, h_ref, sw_ref, out_ref):
        num = af_ref[0] + af_ref[1] + sw_ref[...] * h_ref[...]
        den = ad_ref[0] + ad_ref[1] + sw_ref[...] + EPS
        out_ref[...] = (num / den)[:, :cout]

    grid = (_NROWB,)
    in_specs = [
        pl.BlockSpec((2, _RB, W), lambda i: (0, i, 0)),
        pl.BlockSpec((2, _RB, 1), lambda i: (0, i, 0)),
        pl.BlockSpec((_RB, W), lambda i: (i, 0)),
        pl.BlockSpec((_RB, 1), lambda i: (i, 0)),
    ]
    out_specs = pl.BlockSpec((_RB, cout), lambda i: (i, 0))
    return pl.pallas_call(
        body, out_shape=jax.ShapeDtypeStruct((NNODES, cout), jnp.float32),
        grid=grid, in_specs=in_specs, out_specs=out_specs,
    )(accf, accd, h3, sw3)


_mesh = plsc.VectorSubcoreMesh(core_axis_name="c", subcore_axis_name="s",
                               num_cores=2, num_subcores=16)


@functools.partial(
    pl.kernel,
    out_type=jax.ShapeDtypeStruct((E_PAD,), jnp.float32),
    mesh=_mesh,
    compiler_params=pltpu.CompilerParams(needs_layout_passes=False),
    scratch_types=[
        pltpu.VMEM((CH,), jnp.int32),
        pltpu.VMEM((CH,), jnp.int32),
        pltpu.VMEM((ACC_ROWS,), jnp.float32),
        pltpu.VMEM((ACC_ROWS,), jnp.float32),
        pltpu.VMEM((16,), jnp.float32),
        pltpu.VMEM((CH,), jnp.float32),
    ],
)
def _sc_weights(src_hbm, dst_hbm, asv_hbm, adv_hbm, a_hbm, w_hbm,
                s_ch, d_ch, asv_t, adv_t, a_t, w_ch):
    c = lax.axis_index("c")
    s = lax.axis_index("s")
    wid = c * 16 + s
    base = pl.multiple_of(wid * NE_T, 8)

    pltpu.sync_copy(asv_hbm, asv_t)
    pltpu.sync_copy(adv_hbm, adv_t)
    pltpu.sync_copy(a_hbm, a_t)
    avec = a_t[pl.ds(0, 16)]

    def chunk_body(ch, carry):
        o0 = pl.multiple_of(base + ch * CH, 8)
        pltpu.sync_copy(src_hbm.at[pl.ds(o0, CH)], s_ch)
        pltpu.sync_copy(dst_hbm.at[pl.ds(o0, CH)], d_ch)
        for g in range(CH // 16):
            si = s_ch[pl.ds(g * 16, 16)]
            di = d_ch[pl.ds(g * 16, 16)]
            a_s = plsc.load_gather(asv_t, [si])
            a_d = plsc.load_gather(adv_t, [di])
            t = a_s + a_d
            m = _lrelu(avec + a_d)
            w_ch[pl.ds(g * 16, 16)] = jnp.exp(_lrelu(t) - m)
        pltpu.sync_copy(w_ch, w_hbm.at[pl.ds(o0, CH)])
        return carry

    lax.fori_loop(0, NCH, chunk_body, 0)


@functools.partial(
    pl.kernel,
    out_type=[
        jax.ShapeDtypeStruct((2, ACC_ROWS, W), jnp.float32),
        jax.ShapeDtypeStruct((2, ACC_ROWS), jnp.float32),
    ],
    mesh=_mesh,
    compiler_params=pltpu.CompilerParams(needs_layout_passes=False),
    scratch_types=[
        [pltpu.VMEM((CH,), jnp.int32)] * 4,
        [pltpu.VMEM((CH,), jnp.int32)] * 4,
        [pltpu.VMEM((CH,), jnp.float32)] * 4,
        [pltpu.VMEM((CH, W), jnp.float32)] * 4,
        pltpu.VMEM((RPT,), jnp.float32),
        pltpu.VMEM_SHARED((ACC_ROWS, W), jnp.float32),
        pltpu.VMEM_SHARED((ACC_ROWS,), jnp.float32),
        [pltpu.SemaphoreType.DMA] * 4,
        [pltpu.SemaphoreType.DMA] * 4,
    ],
)
def _sc_edge_phase(src_hbm, dst_hbm, w_hbm, h_hbm, zeros_hbm,
                   outf_hbm, outd_hbm,
                   s_r, d_r, eb_r, rows_r, dbuf, accf, accd, semg_r, sems_r):
    c = lax.axis_index("c")
    s = lax.axis_index("s")
    wid = c * 16 + s
    base = pl.multiple_of(wid * NE_T, 8)
    row0 = pl.multiple_of(s * RPT, 8)

    pltpu.sync_copy(zeros_hbm, rows_r[0])
    for i in range(RPT // CH):
        pltpu.sync_copy(rows_r[0], accf.at[pl.ds(row0 + i * CH, CH)])
    for i in range(RPT // 16):
        dbuf[pl.ds(i * 16, 16)] = jnp.zeros((16,), jnp.float32)
    pltpu.sync_copy(dbuf, accd.at[pl.ds(row0, RPT)])

    plsc.subcore_barrier()

    iota16 = lax.iota(jnp.int32, 16)

    def prep(cidx, r):
        o0 = pl.multiple_of(base + cidx * CH, 8)
        pltpu.sync_copy(src_hbm.at[pl.ds(o0, CH)], s_r[r])
        pltpu.sync_copy(dst_hbm.at[pl.ds(o0, CH)], d_r[r])
        pltpu.sync_copy(w_hbm.at[pl.ds(o0, CH)], eb_r[r])
        pltpu.async_copy(h_hbm.at[s_r[r]], rows_r[r], semg_r[r])

    def process(r):
        pltpu.make_async_copy(h_hbm.at[s_r[r]], rows_r[r], semg_r[r]).wait()

        def scale_body(j):
            js = jnp.full((16,), j, jnp.int32)
            wv = plsc.load_gather(eb_r[r], [js])
            for cc in range(W // 16):
                col = iota16 + (cc * 16)
                v = plsc.load_gather(rows_r[r], [js, col])
                plsc.store_scatter(rows_r[r], [js, col], v * wv)

        plsc.parallel_loop(0, CH, 1, unroll=4, carry=None)(scale_body)
        pltpu.async_copy(rows_r[r], accf.at[d_r[r]], sems_r[r], add=True)
        pltpu.async_copy(eb_r[r], accd.at[d_r[r]], sems_r[r], add=True)

    def wait_scatter(r):
        pltpu.make_async_copy(rows_r[r], accf.at[d_r[r]], sems_r[r]).wait()
        pltpu.make_async_copy(eb_r[r], accd.at[d_r[r]], sems_r[r]).wait()

    for r in range(4):
        prep(r, r)
    ngroups = NCH // 4

    def group_body(k4, carry):
        cidx = k4 * 4

        for r in range(4):
            process(r)

            @pl.when(k4 + 1 < ngroups)
            def _():
                wait_scatter(r)
                prep(cidx + 4 + r, r)

        return carry

    lax.fori_loop(0, ngroups, group_body, 0)
    for r in range(4):
        wait_scatter(r)

    plsc.subcore_barrier()
    for i in range(RPT // CH):
        r0 = pl.multiple_of(row0 + i * CH, 8)
        pltpu.sync_copy(accf.at[pl.ds(r0, CH)], rows_r[0])
        pltpu.sync_copy(rows_r[0], outf_hbm.at[c, pl.ds(r0, CH)])
    pltpu.sync_copy(accd.at[pl.ds(row0, RPT)], dbuf)
    pltpu.sync_copy(dbuf, outd_hbm.at[c, pl.ds(row0, RPT)])


def _pad_nodes(v):
    return jnp.pad(v[:, 0], (0, ACC_ROWS - NNODES))


def kernel(x, edge_index, W1, a_src1, a_dst1, W2, a_src2, a_dst2,
           W3, a_src3, a_dst3):
    src = jnp.pad(edge_index[0].astype(jnp.int32), (0, E_PAD - NEDGES))
    dst = jnp.pad(edge_index[1].astype(jnp.int32), (0, E_PAD - NEDGES),
                  constant_values=ACC_ROWS - 1)
    zeros2d = jnp.zeros((CH, W), jnp.float32)

    h1, asv1, adv1 = _tc_matmul_raw(x, W1, a_src1[None, :], a_dst1[None, :])
    A1, sw1 = _tc_attn(asv1, adv1)
    w1 = _sc_weights(src, dst, _pad_nodes(asv1), _pad_nodes(adv1), A1[0])
    af1, ad1 = _sc_edge_phase(src, dst, w1, h1, zeros2d)

    h2, asv2, adv2 = _tc_matmul_acc(
        af1, ad1[..., None], h1, sw1, W2, a_src2[None, :], a_dst2[None, :])
    A2, sw2 = _tc_attn(asv2, adv2)
    w2 = _sc_weights(src, dst, _pad_nodes(asv2), _pad_nodes(adv2), A2[0])
    af2, ad2 = _sc_edge_phase(src, dst, w2, h2, zeros2d)

    W3p = jnp.pad(W3, ((0, 0), (0, W - W3.shape[1])))
    a_src3p = jnp.pad(a_src3, (0, W - a_src3.shape[0]))
    a_dst3p = jnp.pad(a_dst3, (0, W - a_dst3.shape[0]))
    h3, asv3, adv3 = _tc_matmul_acc(
        af2, ad2[..., None], h2, sw2, W3p, a_src3p[None, :],
        a_dst3p[None, :])
    A3, sw3 = _tc_attn(asv3, adv3)
    w3 = _sc_weights(src, dst, _pad_nodes(asv3), _pad_nodes(adv3), A3[0])
    af3, ad3 = _sc_edge_phase(src, dst, w3, h3, zeros2d)

    return _tc_final(af3, ad3[..., None], h3, sw3, 40)


# speedup vs baseline: 19.2049x; 1.2109x over previous
import functools

import jax
import jax.numpy as jnp
from jax import lax
from jax.experimental import pallas as pl
from jax.experimental.pallas import tpu as pltpu
from jax.experimental.pallas import tpu_sc as plsc

NNODES = 10000
NEDGES = 320000
W = 128
NEG = 0.2
EPS = 1e-16

NTILES = 32
CH = 80
NCH = 128
NE_T = CH * NCH
E_PAD = NE_T * NTILES
ACC_ROWS = 10240
RPT = ACC_ROWS // 16


def _lrelu(x):
    return jnp.where(x >= 0, x, NEG * x)


_RB = 2048
_NROWB = 5


def _tc_matmul_raw(x, Wm, a_s, a_d):
    def body(x_ref, W_ref, as_ref, ad_ref, h_ref, asv_ref, adv_ref):
        h = jnp.dot(x_ref[...], W_ref[...],
                    preferred_element_type=jnp.float32)
        h_ref[...] = h
        asv_ref[...] = jnp.sum(h * as_ref[...], axis=1, keepdims=True)
        adv_ref[...] = jnp.sum(h * ad_ref[...], axis=1, keepdims=True)

    out_shape = [
        jax.ShapeDtypeStruct((NNODES, W), jnp.float32),
        jax.ShapeDtypeStruct((NNODES, 1), jnp.float32),
        jax.ShapeDtypeStruct((NNODES, 1), jnp.float32),
    ]
    return pl.pallas_call(body, out_shape=out_shape)(x, Wm, a_s, a_d)


def _tc_matmul_acc(accf, accd, hp, swp, Wm, a_s, a_d):
    def body(af_ref, adn_ref, hp_ref, swp_ref, W_ref, as_ref, ad_ref,
             h_ref, asv_ref, adv_ref):
        num = af_ref[0] + af_ref[1] + swp_ref[...] * hp_ref[...]
        den = adn_ref[0] + adn_ref[1] + swp_ref[...] + EPS
        o = jnp.maximum(num / den, 0.0)
        h = jnp.dot(o, W_ref[...], preferred_element_type=jnp.float32)
        h_ref[...] = h
        asv_ref[...] = jnp.sum(h * as_ref[...], axis=1, keepdims=True)
        adv_ref[...] = jnp.sum(h * ad_ref[...], axis=1, keepdims=True)

    out_shape = [
        jax.ShapeDtypeStruct((NNODES, W), jnp.float32),
        jax.ShapeDtypeStruct((NNODES, 1), jnp.float32),
        jax.ShapeDtypeStruct((NNODES, 1), jnp.float32),
    ]
    grid = (_NROWB,)
    in_specs = [
        pl.BlockSpec((2, _RB, W), lambda i: (0, i, 0)),
        pl.BlockSpec((2, _RB, 1), lambda i: (0, i, 0)),
        pl.BlockSpec((_RB, W), lambda i: (i, 0)),
        pl.BlockSpec((_RB, 1), lambda i: (i, 0)),
        pl.BlockSpec((W, W), lambda i: (0, 0)),
        pl.BlockSpec((1, W), lambda i: (0, 0)),
        pl.BlockSpec((1, W), lambda i: (0, 0)),
    ]
    out_specs = [
        pl.BlockSpec((_RB, W), lambda i: (i, 0)),
        pl.BlockSpec((_RB, 1), lambda i: (i, 0)),
        pl.BlockSpec((_RB, 1), lambda i: (i, 0)),
    ]
    return pl.pallas_call(body, out_shape=out_shape, grid=grid,
                          in_specs=in_specs, out_specs=out_specs)(
        accf, accd, hp, swp, Wm, a_s, a_d)


def _tc_attn(asv, adv):
    def body(asv_ref, adv_ref, A_ref, sw_ref):
        A = jnp.max(asv_ref[...])
        M = _lrelu(A + adv_ref[...])
        A_ref[...] = jnp.full((1, 16), A, jnp.float32)
        sw_ref[...] = jnp.exp(_lrelu(asv_ref[...] + adv_ref[...]) - M)

    out_shape = [
        jax.ShapeDtypeStruct((1, 16), jnp.float32),
        jax.ShapeDtypeStruct((NNODES, 1), jnp.float32),
    ]
    return pl.pallas_call(body, out_shape=out_shape)(asv, adv)


def _tc_final(accf, accd, h3, sw3, cout):
    def body(af_ref, ad_ref, h_ref, sw_ref, out_ref):
        num = af_ref[0] + af_ref[1] + sw_ref[...] * h_ref[...]
        den = ad_ref[0] + ad_ref[1] + sw_ref[...] + EPS
        out_ref[...] = (num / den)[:, :cout]

    grid = (_NROWB,)
    in_specs = [
        pl.BlockSpec((2, _RB, W), lambda i: (0, i, 0)),
        pl.BlockSpec((2, _RB, 1), lambda i: (0, i, 0)),
        pl.BlockSpec((_RB, W), lambda i: (i, 0)),
        pl.BlockSpec((_RB, 1), lambda i: (i, 0)),
    ]
    out_specs = pl.BlockSpec((_RB, cout), lambda i: (i, 0))
    return pl.pallas_call(
        body, out_shape=jax.ShapeDtypeStruct((NNODES, cout), jnp.float32),
        grid=grid, in_specs=in_specs, out_specs=out_specs,
    )(accf, accd, h3, sw3)


_mesh = plsc.VectorSubcoreMesh(core_axis_name="c", subcore_axis_name="s",
                               num_cores=2, num_subcores=16)


@functools.partial(
    pl.kernel,
    out_type=jax.ShapeDtypeStruct((E_PAD,), jnp.float32),
    mesh=_mesh,
    compiler_params=pltpu.CompilerParams(needs_layout_passes=False),
    scratch_types=[
        pltpu.VMEM((NE_T,), jnp.int32),
        pltpu.VMEM((NE_T,), jnp.int32),
        pltpu.VMEM((ACC_ROWS,), jnp.float32),
        pltpu.VMEM((ACC_ROWS,), jnp.float32),
        pltpu.VMEM((16,), jnp.float32),
        pltpu.VMEM((NE_T,), jnp.float32),
    ],
)
def _sc_weights(src_hbm, dst_hbm, asv_hbm, adv_hbm, a_hbm, w_hbm,
                s_f, d_f, asv_t, adv_t, a_t, w_f):
    c = lax.axis_index("c")
    s = lax.axis_index("s")
    wid = c * 16 + s
    base = pl.multiple_of(wid * NE_T, 8)

    pltpu.sync_copy(asv_hbm, asv_t)
    pltpu.sync_copy(adv_hbm, adv_t)
    pltpu.sync_copy(a_hbm, a_t)
    pltpu.sync_copy(src_hbm.at[pl.ds(base, NE_T)], s_f)
    pltpu.sync_copy(dst_hbm.at[pl.ds(base, NE_T)], d_f)
    avec = a_t[pl.ds(0, 16)]

    def group_body(g):
        o = pl.multiple_of(g * 16, 8)
        si = s_f[pl.ds(o, 16)]
        di = d_f[pl.ds(o, 16)]
        a_s = plsc.load_gather(asv_t, [si])
        a_d = plsc.load_gather(adv_t, [di])
        t = a_s + a_d
        m = _lrelu(avec + a_d)
        w_f[pl.ds(o, 16)] = jnp.exp(_lrelu(t) - m)

    plsc.parallel_loop(0, NE_T // 16, 1, unroll=4, carry=None)(group_body)
    pltpu.sync_copy(w_f, w_hbm.at[pl.ds(base, NE_T)])


@functools.partial(
    pl.kernel,
    out_type=[
        jax.ShapeDtypeStruct((2, ACC_ROWS, W), jnp.float32),
        jax.ShapeDtypeStruct((2, ACC_ROWS), jnp.float32),
    ],
    mesh=_mesh,
    compiler_params=pltpu.CompilerParams(needs_layout_passes=False),
    scratch_types=[
        [pltpu.VMEM((CH,), jnp.int32)] * 4,
        [pltpu.VMEM((CH,), jnp.int32)] * 4,
        [pltpu.VMEM((CH,), jnp.float32)] * 4,
        [pltpu.VMEM((CH, W), jnp.float32)] * 4,
        pltpu.VMEM((RPT,), jnp.float32),
        pltpu.VMEM_SHARED((ACC_ROWS, W), jnp.float32),
        pltpu.VMEM_SHARED((ACC_ROWS,), jnp.float32),
        [pltpu.SemaphoreType.DMA] * 4,
        [pltpu.SemaphoreType.DMA] * 4,
    ],
)
def _sc_edge_phase(src_hbm, dst_hbm, w_hbm, h_hbm, zeros_hbm,
                   outf_hbm, outd_hbm,
                   s_r, d_r, eb_r, rows_r, dbuf, accf, accd, semg_r, sems_r):
    c = lax.axis_index("c")
    s = lax.axis_index("s")
    wid = c * 16 + s
    base = pl.multiple_of(wid * NE_T, 8)
    row0 = pl.multiple_of(s * RPT, 8)

    pltpu.sync_copy(zeros_hbm, rows_r[0])
    for i in range(RPT // CH):
        pltpu.sync_copy(rows_r[0], accf.at[pl.ds(row0 + i * CH, CH)])
    for i in range(RPT // 16):
        dbuf[pl.ds(i * 16, 16)] = jnp.zeros((16,), jnp.float32)
    pltpu.sync_copy(dbuf, accd.at[pl.ds(row0, RPT)])

    plsc.subcore_barrier()

    iota16 = lax.iota(jnp.int32, 16)

    def prep(cidx, r):
        o0 = pl.multiple_of(base + cidx * CH, 8)
        pltpu.sync_copy(src_hbm.at[pl.ds(o0, CH)], s_r[r])
        pltpu.sync_copy(dst_hbm.at[pl.ds(o0, CH)], d_r[r])
        pltpu.sync_copy(w_hbm.at[pl.ds(o0, CH)], eb_r[r])
        pltpu.async_copy(h_hbm.at[s_r[r]], rows_r[r], semg_r[r])

    def process(r):
        pltpu.make_async_copy(h_hbm.at[s_r[r]], rows_r[r], semg_r[r]).wait()

        def scale_body(j):
            js = jnp.full((16,), j, jnp.int32)
            wv = plsc.load_gather(eb_r[r], [js])
            for cc in range(W // 16):
                col = iota16 + (cc * 16)
                v = plsc.load_gather(rows_r[r], [js, col])
                plsc.store_scatter(rows_r[r], [js, col], v * wv)

        plsc.parallel_loop(0, CH, 1, unroll=4, carry=None)(scale_body)
        pltpu.async_copy(rows_r[r], accf.at[d_r[r]], sems_r[r], add=True)
        pltpu.async_copy(eb_r[r], accd.at[d_r[r]], sems_r[r], add=True)

    def wait_scatter(r):
        pltpu.make_async_copy(rows_r[r], accf.at[d_r[r]], sems_r[r]).wait()
        pltpu.make_async_copy(eb_r[r], accd.at[d_r[r]], sems_r[r]).wait()

    for r in range(4):
        prep(r, r)
    ngroups = NCH // 4

    def group_body(k4, carry):
        cidx = k4 * 4

        for r in range(4):
            process(r)

            @pl.when(k4 + 1 < ngroups)
            def _():
                wait_scatter(r)
                prep(cidx + 4 + r, r)

        return carry

    lax.fori_loop(0, ngroups, group_body, 0)
    for r in range(4):
        wait_scatter(r)

    plsc.subcore_barrier()
    for i in range(RPT // CH):
        r0 = pl.multiple_of(row0 + i * CH, 8)
        pltpu.sync_copy(accf.at[pl.ds(r0, CH)], rows_r[0])
        pltpu.sync_copy(rows_r[0], outf_hbm.at[c, pl.ds(r0, CH)])
    pltpu.sync_copy(accd.at[pl.ds(row0, RPT)], dbuf)
    pltpu.sync_copy(dbuf, outd_hbm.at[c, pl.ds(row0, RPT)])


def _pad_nodes(v):
    return jnp.pad(v[:, 0], (0, ACC_ROWS - NNODES))


def kernel(x, edge_index, W1, a_src1, a_dst1, W2, a_src2, a_dst2,
           W3, a_src3, a_dst3):
    src = jnp.pad(edge_index[0].astype(jnp.int32), (0, E_PAD - NEDGES))
    dst = jnp.pad(edge_index[1].astype(jnp.int32), (0, E_PAD - NEDGES),
                  constant_values=ACC_ROWS - 1)
    zeros2d = jnp.zeros((CH, W), jnp.float32)

    h1, asv1, adv1 = _tc_matmul_raw(x, W1, a_src1[None, :], a_dst1[None, :])
    A1, sw1 = _tc_attn(asv1, adv1)
    w1 = _sc_weights(src, dst, _pad_nodes(asv1), _pad_nodes(adv1), A1[0])
    af1, ad1 = _sc_edge_phase(src, dst, w1, h1, zeros2d)

    h2, asv2, adv2 = _tc_matmul_acc(
        af1, ad1[..., None], h1, sw1, W2, a_src2[None, :], a_dst2[None, :])
    A2, sw2 = _tc_attn(asv2, adv2)
    w2 = _sc_weights(src, dst, _pad_nodes(asv2), _pad_nodes(adv2), A2[0])
    af2, ad2 = _sc_edge_phase(src, dst, w2, h2, zeros2d)

    W3p = jnp.pad(W3, ((0, 0), (0, W - W3.shape[1])))
    a_src3p = jnp.pad(a_src3, (0, W - a_src3.shape[0]))
    a_dst3p = jnp.pad(a_dst3, (0, W - a_dst3.shape[0]))
    h3, asv3, adv3 = _tc_matmul_acc(
        af2, ad2[..., None], h2, sw2, W3p, a_src3p[None, :],
        a_dst3p[None, :])
    A3, sw3 = _tc_attn(asv3, adv3)
    w3 = _sc_weights(src, dst, _pad_nodes(asv3), _pad_nodes(adv3), A3[0])
    af3, ad3 = _sc_edge_phase(src, dst, w3, h3, zeros2d)

    return _tc_final(af3, ad3[..., None], h3, sw3, 40)


# speedup vs baseline: 19.2120x; 1.0004x over previous
import functools

import jax
import jax.numpy as jnp
from jax import lax
from jax.experimental import pallas as pl
from jax.experimental.pallas import tpu as pltpu
from jax.experimental.pallas import tpu_sc as plsc

NNODES = 10000
NEDGES = 320000
W = 128
NEG = 0.2
EPS = 1e-16

NTILES = 32
CH = 80
NCH = 128
NE_T = CH * NCH
E_PAD = NE_T * NTILES
ACC_ROWS = 10240
RPT = ACC_ROWS // 16


def _lrelu(x):
    return jnp.where(x >= 0, x, NEG * x)


_RB = 2048
_NROWB = 5


def _tc_matmul_raw(x, Wm, a_s, a_d):
    def body(x_ref, W_ref, as_ref, ad_ref, h_ref, asv_ref, adv_ref):
        h = jnp.dot(x_ref[...], W_ref[...],
                    preferred_element_type=jnp.float32)
        h_ref[...] = h
        asv_ref[...] = jnp.sum(h * as_ref[...], axis=1, keepdims=True)
        adv_ref[...] = jnp.sum(h * ad_ref[...], axis=1, keepdims=True)

    out_shape = [
        jax.ShapeDtypeStruct((NNODES, W), jnp.float32),
        jax.ShapeDtypeStruct((NNODES, 1), jnp.float32),
        jax.ShapeDtypeStruct((NNODES, 1), jnp.float32),
    ]
    return pl.pallas_call(body, out_shape=out_shape)(x, Wm, a_s, a_d)


def _tc_matmul_acc(accf, accd, hp, swp, Wm, a_s, a_d):
    def body(af_ref, adn_ref, hp_ref, swp_ref, W_ref, as_ref, ad_ref,
             h_ref, asv_ref, adv_ref):
        num = af_ref[0] + af_ref[1] + swp_ref[...] * hp_ref[...]
        den = adn_ref[0] + adn_ref[1] + swp_ref[...] + EPS
        o = jnp.maximum(num / den, 0.0)
        h = jnp.dot(o, W_ref[...], preferred_element_type=jnp.float32)
        h_ref[...] = h
        asv_ref[...] = jnp.sum(h * as_ref[...], axis=1, keepdims=True)
        adv_ref[...] = jnp.sum(h * ad_ref[...], axis=1, keepdims=True)

    out_shape = [
        jax.ShapeDtypeStruct((NNODES, W), jnp.float32),
        jax.ShapeDtypeStruct((NNODES, 1), jnp.float32),
        jax.ShapeDtypeStruct((NNODES, 1), jnp.float32),
    ]
    grid = (_NROWB,)
    in_specs = [
        pl.BlockSpec((2, _RB, W), lambda i: (0, i, 0)),
        pl.BlockSpec((2, _RB, 1), lambda i: (0, i, 0)),
        pl.BlockSpec((_RB, W), lambda i: (i, 0)),
        pl.BlockSpec((_RB, 1), lambda i: (i, 0)),
        pl.BlockSpec((W, W), lambda i: (0, 0)),
        pl.BlockSpec((1, W), lambda i: (0, 0)),
        pl.BlockSpec((1, W), lambda i: (0, 0)),
    ]
    out_specs = [
        pl.BlockSpec((_RB, W), lambda i: (i, 0)),
        pl.BlockSpec((_RB, 1), lambda i: (i, 0)),
        pl.BlockSpec((_RB, 1), lambda i: (i, 0)),
    ]
    return pl.pallas_call(body, out_shape=out_shape, grid=grid,
                          in_specs=in_specs, out_specs=out_specs)(
        accf, accd, hp, swp, Wm, a_s, a_d)


def _tc_attn(asv, adv):
    def body(asv_ref, adv_ref, A_ref, sw_ref):
        A = jnp.max(asv_ref[...])
        M = _lrelu(A + adv_ref[...])
        A_ref[...] = jnp.full((1, 16), A, jnp.float32)
        sw_ref[...] = jnp.exp(_lrelu(asv_ref[...] + adv_ref[...]) - M)

    out_shape = [
        jax.ShapeDtypeStruct((1, 16), jnp.float32),
        jax.ShapeDtypeStruct((NNODES, 1), jnp.float32),
    ]
    return pl.pallas_call(body, out_shape=out_shape)(asv, adv)


def _tc_final(accf, accd, h3, sw3, cout):
    def body(af_ref, ad_ref, h_ref, sw_ref, out_ref):
        num = af_ref[0] + af_ref[1] + sw_ref[...] * h_ref[...]
        den = ad_ref[0] + ad_ref[1] + sw_ref[...] + EPS
        out_ref[...] = (num / den)[:, :cout]

    grid = (_NROWB,)
    in_specs = [
        pl.BlockSpec((2, _RB, W), lambda i: (0, i, 0)),
        pl.BlockSpec((2, _RB, 1), lambda i: (0, i, 0)),
        pl.BlockSpec((_RB, W), lambda i: (i, 0)),
        pl.BlockSpec((_RB, 1), lambda i: (i, 0)),
    ]
    out_specs = pl.BlockSpec((_RB, cout), lambda i: (i, 0))
    return pl.pallas_call(
        body, out_shape=jax.ShapeDtypeStruct((NNODES, cout), jnp.float32),
        grid=grid, in_specs=in_specs, out_specs=out_specs,
    )(accf, accd, h3, sw3)


_mesh = plsc.VectorSubcoreMesh(core_axis_name="c", subcore_axis_name="s",
                               num_cores=2, num_subcores=16)


@functools.partial(
    pl.kernel,
    out_type=jax.ShapeDtypeStruct((E_PAD,), jnp.float32),
    mesh=_mesh,
    compiler_params=pltpu.CompilerParams(needs_layout_passes=False),
    scratch_types=[
        pltpu.VMEM((NE_T,), jnp.int32),
        pltpu.VMEM((NE_T,), jnp.int32),
        pltpu.VMEM((ACC_ROWS,), jnp.float32),
        pltpu.VMEM((ACC_ROWS,), jnp.float32),
        pltpu.VMEM((16,), jnp.float32),
        pltpu.VMEM((NE_T,), jnp.float32),
    ],
)
def _sc_weights(src_hbm, dst_hbm, asv_hbm, adv_hbm, a_hbm, w_hbm,
                s_f, d_f, asv_t, adv_t, a_t, w_f):
    c = lax.axis_index("c")
    s = lax.axis_index("s")
    wid = c * 16 + s
    base = pl.multiple_of(wid * NE_T, 8)

    pltpu.sync_copy(asv_hbm, asv_t)
    pltpu.sync_copy(adv_hbm, adv_t)
    pltpu.sync_copy(a_hbm, a_t)
    pltpu.sync_copy(src_hbm.at[pl.ds(base, NE_T)], s_f)
    pltpu.sync_copy(dst_hbm.at[pl.ds(base, NE_T)], d_f)
    avec = a_t[pl.ds(0, 16)]

    def group_body(g):
        o = pl.multiple_of(g * 16, 8)
        si = s_f[pl.ds(o, 16)]
        di = d_f[pl.ds(o, 16)]
        a_s = plsc.load_gather(asv_t, [si])
        a_d = plsc.load_gather(adv_t, [di])
        t = a_s + a_d
        m = _lrelu(avec + a_d)
        w_f[pl.ds(o, 16)] = jnp.exp(_lrelu(t) - m)

    plsc.parallel_loop(0, NE_T // 16, 1, unroll=4, carry=None)(group_body)
    pltpu.sync_copy(w_f, w_hbm.at[pl.ds(base, NE_T)])


@functools.partial(
    pl.kernel,
    out_type=[
        jax.ShapeDtypeStruct((2, ACC_ROWS, W), jnp.float32),
        jax.ShapeDtypeStruct((2, ACC_ROWS), jnp.float32),
    ],
    mesh=_mesh,
    compiler_params=pltpu.CompilerParams(needs_layout_passes=False),
    scratch_types=[
        [pltpu.VMEM((CH,), jnp.int32)] * 4,
        [pltpu.VMEM((CH,), jnp.int32)] * 4,
        [pltpu.VMEM((CH,), jnp.float32)] * 4,
        [pltpu.VMEM((CH, W), jnp.float32)] * 4,
        pltpu.VMEM((RPT,), jnp.float32),
        pltpu.VMEM_SHARED((ACC_ROWS, W), jnp.float32),
        pltpu.VMEM_SHARED((ACC_ROWS,), jnp.float32),
        [pltpu.SemaphoreType.DMA] * 4,
        [pltpu.SemaphoreType.DMA] * 4,
    ],
)
def _sc_edge_phase(src_hbm, dst_hbm, w_hbm, h_hbm, zeros_hbm,
                   outf_hbm, outd_hbm,
                   s_r, d_r, eb_r, rows_r, dbuf, accf, accd, semg_r, sems_r):
    c = lax.axis_index("c")
    s = lax.axis_index("s")
    wid = c * 16 + s
    base = pl.multiple_of(wid * NE_T, 8)
    row0 = pl.multiple_of(s * RPT, 8)

    pltpu.sync_copy(zeros_hbm, rows_r[0])
    for i in range(RPT // CH):
        pltpu.sync_copy(rows_r[0], accf.at[pl.ds(row0 + i * CH, CH)])
    for i in range(RPT // 16):
        dbuf[pl.ds(i * 16, 16)] = jnp.zeros((16,), jnp.float32)
    pltpu.sync_copy(dbuf, accd.at[pl.ds(row0, RPT)])

    plsc.subcore_barrier()

    iota16 = lax.iota(jnp.int32, 16)

    def prep(cidx, r):
        o0 = pl.multiple_of(base + cidx * CH, 8)
        pltpu.sync_copy(src_hbm.at[pl.ds(o0, CH)], s_r[r])
        pltpu.sync_copy(dst_hbm.at[pl.ds(o0, CH)], d_r[r])
        pltpu.sync_copy(w_hbm.at[pl.ds(o0, CH)], eb_r[r])
        hh = CH // 2
        pltpu.async_copy(h_hbm.at[s_r[r].at[pl.ds(0, hh)]],
                         rows_r[r].at[pl.ds(0, hh)], semg_r[r])
        pltpu.async_copy(h_hbm.at[s_r[r].at[pl.ds(hh, hh)]],
                         rows_r[r].at[pl.ds(hh, hh)], semg_r[r])

    def process(r):
        hh = CH // 2
        pltpu.make_async_copy(h_hbm.at[s_r[r].at[pl.ds(0, hh)]],
                              rows_r[r].at[pl.ds(0, hh)], semg_r[r]).wait()
        pltpu.make_async_copy(h_hbm.at[s_r[r].at[pl.ds(hh, hh)]],
                              rows_r[r].at[pl.ds(hh, hh)], semg_r[r]).wait()

        def scale_body(j):
            js = jnp.full((16,), j, jnp.int32)
            wv = plsc.load_gather(eb_r[r], [js])
            for cc in range(W // 16):
                col = iota16 + (cc * 16)
                v = plsc.load_gather(rows_r[r], [js, col])
                plsc.store_scatter(rows_r[r], [js, col], v * wv)

        plsc.parallel_loop(0, CH, 1, unroll=4, carry=None)(scale_body)
        pltpu.async_copy(rows_r[r], accf.at[d_r[r]], sems_r[r], add=True)
        pltpu.async_copy(eb_r[r], accd.at[d_r[r]], sems_r[r], add=True)

    def wait_scatter(r):
        pltpu.make_async_copy(rows_r[r], accf.at[d_r[r]], sems_r[r]).wait()
        pltpu.make_async_copy(eb_r[r], accd.at[d_r[r]], sems_r[r]).wait()

    for r in range(4):
        prep(r, r)
    ngroups = NCH // 4

    def group_body(k4, carry):
        cidx = k4 * 4

        for r in range(4):
            process(r)

            @pl.when(k4 + 1 < ngroups)
            def _():
                wait_scatter(r)
                prep(cidx + 4 + r, r)

        return carry

    lax.fori_loop(0, ngroups, group_body, 0)
    for r in range(4):
        wait_scatter(r)

    plsc.subcore_barrier()
    for i in range(RPT // CH):
        r0 = pl.multiple_of(row0 + i * CH, 8)
        pltpu.sync_copy(accf.at[pl.ds(r0, CH)], rows_r[0])
        pltpu.sync_copy(rows_r[0], outf_hbm.at[c, pl.ds(r0, CH)])
    pltpu.sync_copy(accd.at[pl.ds(row0, RPT)], dbuf)
    pltpu.sync_copy(dbuf, outd_hbm.at[c, pl.ds(row0, RPT)])


def _pad_nodes(v):
    return jnp.pad(v[:, 0], (0, ACC_ROWS - NNODES))


def kernel(x, edge_index, W1, a_src1, a_dst1, W2, a_src2, a_dst2,
           W3, a_src3, a_dst3):
    src = jnp.pad(edge_index[0].astype(jnp.int32), (0, E_PAD - NEDGES))
    dst = jnp.pad(edge_index[1].astype(jnp.int32), (0, E_PAD - NEDGES),
                  constant_values=ACC_ROWS - 1)
    zeros2d = jnp.zeros((CH, W), jnp.float32)

    h1, asv1, adv1 = _tc_matmul_raw(x, W1, a_src1[None, :], a_dst1[None, :])
    A1, sw1 = _tc_attn(asv1, adv1)
    w1 = _sc_weights(src, dst, _pad_nodes(asv1), _pad_nodes(adv1), A1[0])
    af1, ad1 = _sc_edge_phase(src, dst, w1, h1, zeros2d)

    h2, asv2, adv2 = _tc_matmul_acc(
        af1, ad1[..., None], h1, sw1, W2, a_src2[None, :], a_dst2[None, :])
    A2, sw2 = _tc_attn(asv2, adv2)
    w2 = _sc_weights(src, dst, _pad_nodes(asv2), _pad_nodes(adv2), A2[0])
    af2, ad2 = _sc_edge_phase(src, dst, w2, h2, zeros2d)

    W3p = jnp.pad(W3, ((0, 0), (0, W - W3.shape[1])))
    a_src3p = jnp.pad(a_src3, (0, W - a_src3.shape[0]))
    a_dst3p = jnp.pad(a_dst3, (0, W - a_dst3.shape[0]))
    h3, asv3, adv3 = _tc_matmul_acc(
        af2, ad2[..., None], h2, sw2, W3p, a_src3p[None, :],
        a_dst3p[None, :])
    A3, sw3 = _tc_attn(asv3, adv3)
    w3 = _sc_weights(src, dst, _pad_nodes(asv3), _pad_nodes(adv3), A3[0])
    af3, ad3 = _sc_edge_phase(src, dst, w3, h3, zeros2d)

    return _tc_final(af3, ad3[..., None], h3, sw3, 40)
